# Initial kernel scaffold; baseline (speedup 1.0000x reference)
import jax
import jax.numpy as jnp
from jax.experimental import pallas as pl


def kernel(x, adj, src, tgt, Msrc, Mtgt, W_in_w, W_in_b, a_w, a_b, W_out_w, W_out_b):
    raise NotImplementedError("write your pallas kernel here")


# speedup vs baseline: 3.7242x; 3.7242x over previous
import functools

import jax
import jax.numpy as jnp
from jax import lax
from jax.experimental import pallas as pl
from jax.experimental.pallas import tpu as pltpu
from jax.experimental.pallas import tpu_sc as plsc

N = 4096
E = 32768
D = 128
EPS = 1e-6

NC = 2
NS = 16
L = 16
NW = NC * NS
EPW = E // NW
CHUNK = 128
NCHUNK = EPW // CHUNK
GROUPS = CHUNK // L
NPS = N // NS


def _mm_body(x_ref, w_ref, b_ref, u_ref, v_ref, wo_ref):
    y = jnp.dot(x_ref[...], w_ref[...], preferred_element_type=jnp.float32)
    y = y + b_ref[...]
    u_ref[...] = y[:, :D]
    v_ref[...] = y[:, D:2 * D]
    wo_ref[...] = y[:, 2 * D:]


def _fin_body(w_ref, s0_ref, s1_ref, o_ref):
    s = s0_ref[...] + s1_ref[...]
    scale = s / (s + EPS)
    y = w_ref[...] * scale
    o_ref[...] = jnp.where(y >= 0, y, 0.01 * y)


def _edge_body(u_hbm, v_hbm, src_hbm, tgt_hbm, a_hbm, ab_hbm, out0, out1,
               src_v, tgt_v, u_rows, v_rows, exp_v, a_v, ab_v, zero_v,
               seg_sh, sem_u, sem_v):
    cid = lax.axis_index("c")
    sid = lax.axis_index("s")
    wid = cid * NS + sid

    for i in range(NPS // L):
        zero_v[pl.ds(i * L, L)] = jnp.zeros((L,), jnp.float32)
    pltpu.sync_copy(zero_v, seg_sh.at[pl.ds(sid * NPS, NPS)])

    pltpu.sync_copy(a_hbm, a_v)
    pltpu.sync_copy(ab_hbm, ab_v)
    ab_vec = ab_v[...]

    plsc.subcore_barrier()

    row_ids = [lax.iota(jnp.int32, L) + g * L for g in range(GROUPS)]

    for c in range(NCHUNK):
        base = wid * EPW + c * CHUNK
        pltpu.sync_copy(src_hbm.at[pl.ds(base, CHUNK)], src_v)
        pltpu.sync_copy(tgt_hbm.at[pl.ds(base, CHUNK)], tgt_v)
        cp_u = pltpu.async_copy(u_hbm.at[src_v], u_rows, sem_u)
        cp_v = pltpu.async_copy(v_hbm.at[tgt_v], v_rows, sem_v)
        cp_u.wait()
        cp_v.wait()

        def kbody(k, accs):
            colv = jnp.full((L,), k, jnp.int32)
            ak = plsc.load_gather(a_v, [colv])
            new = []
            for g in range(GROUPS):
                ug = plsc.load_gather(u_rows, [row_ids[g], colv])
                vg = plsc.load_gather(v_rows, [row_ids[g], colv])
                z = ug + vg
                lr = jnp.maximum(z, 0.01 * z)
                new.append(accs[g] + ak * lr)
            return tuple(new)

        accs = lax.fori_loop(
            0, D, kbody,
            tuple(jnp.zeros((L,), jnp.float32) for _ in range(GROUPS)))
        for g in range(GROUPS):
            exp_v[pl.ds(g * L, L)] = jnp.exp(accs[g] + ab_vec)

        pltpu.sync_copy(exp_v, seg_sh.at[tgt_v], add=True)

    plsc.subcore_barrier()

    sl = pl.ds(sid * NPS, NPS)

    @pl.when(cid == 0)
    def _():
        pltpu.sync_copy(seg_sh.at[sl], out0.at[sl])

    @pl.when(cid == 1)
    def _():
        pltpu.sync_copy(seg_sh.at[sl], out1.at[sl])


@functools.lru_cache(maxsize=1)
def _edge_kernel():
    mesh = plsc.VectorSubcoreMesh(
        core_axis_name="c", subcore_axis_name="s",
        num_cores=NC, num_subcores=NS)
    return pl.kernel(
        _edge_body,
        compiler_params=pltpu.CompilerParams(needs_layout_passes=False),
        out_type=(jax.ShapeDtypeStruct((N,), jnp.float32),
                  jax.ShapeDtypeStruct((N,), jnp.float32)),
        mesh=mesh,
        scratch_types=[
            pltpu.VMEM((CHUNK,), jnp.int32),
            pltpu.VMEM((CHUNK,), jnp.int32),
            pltpu.VMEM((CHUNK, D), jnp.float32),
            pltpu.VMEM((CHUNK, D), jnp.float32),
            pltpu.VMEM((CHUNK,), jnp.float32),
            pltpu.VMEM((D,), jnp.float32),
            pltpu.VMEM((L,), jnp.float32),
            pltpu.VMEM((NPS,), jnp.float32),
            pltpu.VMEM_SHARED((N,), jnp.float32),
            pltpu.SemaphoreType.DMA,
            pltpu.SemaphoreType.DMA,
        ],
    )


def kernel(x, adj, src, tgt, Msrc, Mtgt, W_in_w, W_in_b, a_w, a_b, W_out_w, W_out_b):
    wcat = jnp.concatenate(
        [W_in_w[:, :D].T, W_in_w[:, D:].T, W_out_w.T], axis=1)
    bcat = jnp.concatenate(
        [jnp.zeros((D,), jnp.float32), W_in_b, W_out_b])[None, :]

    blk = 512
    u, v, w = pl.pallas_call(
        _mm_body,
        grid=(N // blk,),
        in_specs=[
            pl.BlockSpec((blk, D), lambda i: (i, 0)),
            pl.BlockSpec((D, 3 * D), lambda i: (0, 0)),
            pl.BlockSpec((1, 3 * D), lambda i: (0, 0)),
        ],
        out_specs=[
            pl.BlockSpec((blk, D), lambda i: (i, 0)),
            pl.BlockSpec((blk, D), lambda i: (i, 0)),
            pl.BlockSpec((blk, D), lambda i: (i, 0)),
        ],
        out_shape=[
            jax.ShapeDtypeStruct((N, D), jnp.float32),
            jax.ShapeDtypeStruct((N, D), jnp.float32),
            jax.ShapeDtypeStruct((N, D), jnp.float32),
        ],
    )(x, wcat, bcat)

    src32 = src.astype(jnp.int32)
    tgt32 = tgt.astype(jnp.int32)
    a_vec = a_w.reshape(D)
    ab_vec = jnp.broadcast_to(a_b.reshape(1), (L,))

    seg0, seg1 = _edge_kernel()(u, v, src32, tgt32, a_vec, ab_vec)

    out = pl.pallas_call(
        _fin_body,
        grid=(N // blk,),
        in_specs=[
            pl.BlockSpec((blk, D), lambda i: (i, 0)),
            pl.BlockSpec((blk, 1), lambda i: (i, 0)),
            pl.BlockSpec((blk, 1), lambda i: (i, 0)),
        ],
        out_specs=pl.BlockSpec((blk, D), lambda i: (i, 0)),
        out_shape=jax.ShapeDtypeStruct((N, D), jnp.float32),
    )(w, seg0.reshape(N, 1), seg1.reshape(N, 1))
    return out
